# baseline (speedup 1.0000x reference)
import jax
import jax.numpy as jnp
from jax import lax
from jax.experimental import pallas as pl

N_TOK = 32768
D_MODEL = 768
N_EXP = 64
_BT = 4096
_H = _BT // 2


def _gate_body(xa_ref, xb_ref, w_ref, idxa_ref, gatea_ref, idxb_ref, gateb_ref):
    w = w_ref[...]

    def half(x_ref, idx_ref, gate_ref):
        logits = lax.dot_general(
            w, x_ref[...],
            (((1,), (1,)), ((), ())),
            preferred_element_type=jnp.float32,
        )
        m = jnp.max(logits, axis=0, keepdims=True)
        ii = lax.broadcasted_iota(jnp.int32, (N_EXP, _H), 0)
        cand = jnp.where(logits == m, ii, N_EXP)
        idx_ref[...] = jnp.min(cand, axis=0, keepdims=True)
        s = jnp.sum(jnp.exp(logits - m), axis=0, keepdims=True)
        gate_ref[...] = 1.0 / s

    half(xa_ref, idxa_ref, gatea_ref)
    half(xb_ref, idxb_ref, gateb_ref)


def _interleave(a, b):
    a2 = a.reshape(-1, _H)
    b2 = b.reshape(-1, _H)
    return jnp.stack([a2, b2], axis=1).reshape(N_TOK)


def kernel(x, W):
    idxa, gatea, idxb, gateb = pl.pallas_call(
        _gate_body,
        grid=(N_TOK // _BT,),
        in_specs=[
            pl.BlockSpec((_H, D_MODEL), lambda i: (2 * i, 0)),
            pl.BlockSpec((_H, D_MODEL), lambda i: (2 * i + 1, 0)),
            pl.BlockSpec((N_EXP, D_MODEL), lambda i: (0, 0)),
        ],
        out_specs=[
            pl.BlockSpec((1, _H), lambda i: (0, i)),
            pl.BlockSpec((1, _H), lambda i: (0, i)),
            pl.BlockSpec((1, _H), lambda i: (0, i)),
            pl.BlockSpec((1, _H), lambda i: (0, i)),
        ],
        out_shape=[
            jax.ShapeDtypeStruct((1, N_TOK // 2), jnp.int32),
            jax.ShapeDtypeStruct((1, N_TOK // 2), jnp.float32),
            jax.ShapeDtypeStruct((1, N_TOK // 2), jnp.int32),
            jax.ShapeDtypeStruct((1, N_TOK // 2), jnp.float32),
        ],
    )(x, x, W)
    expert_indices = _interleave(idxa, idxb)
    expert_gates = _interleave(gatea, gateb)
    load_balance_loss = jnp.zeros((), jnp.float32)
    return (expert_indices, expert_gates, load_balance_loss)
